# baseline (speedup 1.0000x reference)
import jax
import jax.numpy as jnp
from jax.experimental import pallas as pl
from jax.experimental.pallas import tpu as pltpu

NGEOM = 200
BSIZE = 64
NVALS = 2000000


def _dense1_body(S_ref, G_ref, rho_ref, qn_ref, phiS_ref, H_ref,
                 fockp_ref, ener2_ref):
    S = S_ref[0]
    G = G_ref[0]
    rho_in = rho_ref[0]
    qn = qn_ref[0]
    phiS = phiS_ref[0]
    H = H_ref[0]
    qbasis = rho_in * S
    GOP = jnp.sum(qbasis, axis=1, keepdims=True)
    dQ = qn - GOP
    ep = jnp.dot(G, dQ, preferred_element_type=jnp.float32)
    couMat = -0.5 * S * (ep + ep.T)
    F = H + couMat
    FP = jnp.dot(F, phiS, preferred_element_type=jnp.float32)
    fockp = jnp.dot(phiS.T, FP, preferred_element_type=jnp.float32)
    fockp = 0.5 * (fockp + fockp.T)
    fockp_ref[0] = fockp
    ener2_ref[0, 0, :] = jnp.full((128,), 0.5, jnp.float32) * jnp.sum(dQ * ep)


def _dense2_body(phiS_ref, occ_ref, W_ref, H_ref, ener1_ref):
    phiS = phiS_ref[0]
    occ = occ_ref[0]
    W = W_ref[0]
    H = H_ref[0]
    orb = jnp.dot(phiS, W, preferred_element_type=jnp.float32)
    orb_f = occ * orb
    rho = 2.0 * jnp.dot(orb_f, orb_f.T, preferred_element_type=jnp.float32)
    ener1_ref[0, 0, :] = jnp.full((128,), 1.0, jnp.float32) * jnp.sum(rho * H)


def _mat_spec():
    return pl.BlockSpec((1, BSIZE, BSIZE), lambda g: (g, 0, 0))


def _vec_spec():
    return pl.BlockSpec((1, BSIZE, 1), lambda g: (g, 0, 0))


def _scalar_spec():
    return pl.BlockSpec((1, 1, 128), lambda g: (g, 0, 0))


def kernel(x, net_base, rot_tensor, S, G, rho_in, qneutral, phiS, occ_mask,
           idx_x, gather_rot, gather_oper, gather_rep, seg_rep):
    net_vals = net_base.at[idx_x].set(x)
    vals = net_vals[gather_rot].reshape((-1, 3))[:, :, None]
    rot_out_temp = jnp.matmul(rot_tensor, vals)[:, :, 0]
    rot_out = jnp.concatenate(
        [jnp.array([0.0, 1.0], dtype=jnp.float32), rot_out_temp.ravel()])
    H = rot_out[gather_oper]

    fockp, ener2 = pl.pallas_call(
        _dense1_body,
        grid=(NGEOM,),
        in_specs=[_mat_spec(), _mat_spec(), _mat_spec(), _vec_spec(),
                  _mat_spec(), _mat_spec()],
        out_specs=[_mat_spec(), _scalar_spec()],
        out_shape=[
            jax.ShapeDtypeStruct((NGEOM, BSIZE, BSIZE), jnp.float32),
            jax.ShapeDtypeStruct((NGEOM, 1, 128), jnp.float32),
        ],
    )(S, G, rho_in, qneutral, phiS, H)

    _, W = jnp.linalg.eigh(fockp)

    ener1 = pl.pallas_call(
        _dense2_body,
        grid=(NGEOM,),
        in_specs=[_mat_spec(), _mat_spec(), _mat_spec(), _mat_spec()],
        out_specs=_scalar_spec(),
        out_shape=jax.ShapeDtypeStruct((NGEOM, 1, 128), jnp.float32),
    )(phiS, occ_mask, W, H)

    Erep = jax.ops.segment_sum(net_vals[gather_rep], seg_rep,
                               num_segments=NGEOM)
    return ener1[:, 0, 0] + ener2[:, 0, 0] + Erep


# speedup vs baseline: 2.3549x; 2.3549x over previous
import functools

import jax
import jax.numpy as jnp
from jax import lax
from jax.experimental import pallas as pl
from jax.experimental.pallas import tpu as pltpu

NGEOM = 200
BSIZE = 64
NVALS = 2000000
BGEOM = 8
NSWEEP = 5


def _dgT00(a, b):
    return lax.dot_general(a, b, (((0,), (0,)), ((), ())),
                           preferred_element_type=jnp.float32)


def _dg11(a, b):
    return lax.dot_general(a, b, (((1,), (1,)), ((), ())),
                           preferred_element_type=jnp.float32)


def _coeffs(app, aqq, apq):
    zero = apq == 0.0
    apq_s = jnp.where(zero, 1.0, apq)
    tau = (aqq - app) / (2.0 * apq_s)
    t = jnp.sign(tau) / (jnp.abs(tau) + jnp.sqrt(tau * tau + 1.0))
    t = jnp.where(tau == 0.0, 1.0, t)
    c = lax.rsqrt(t * t + 1.0)
    s = t * c
    c = jnp.where(zero, 1.0, c)
    s = jnp.where(zero, 0.0, s)
    return c, s


def _perm_rows(A):
    return jnp.concatenate([A[:, 0:1], A[:, 32:33], A[:, 1:31],
                            A[:, 33:64], A[:, 31:32]], axis=1)


def _perm_cols(A):
    return jnp.concatenate([A[:, :, 0:1], A[:, :, 32:33], A[:, :, 1:31],
                            A[:, :, 33:64], A[:, :, 31:32]], axis=2)


def _masks():
    ks = lax.broadcasted_iota(jnp.int32, (1, BSIZE, BSIZE), 1)
    js = lax.broadcasted_iota(jnp.int32, (1, BSIZE, BSIZE), 2)
    eye = ks == js
    off = js == ks + 32
    return eye, off


def _jacobi_step(_, carry):
    A, V = carry
    eye, off = _masks()
    zf = jnp.zeros_like(A)
    Aeye = jnp.where(eye, A, zf)
    Aoff = jnp.where(off, A, zf)
    dv_s = jnp.sum(Aeye, axis=2, keepdims=True)
    ov_s = jnp.sum(Aoff, axis=2, keepdims=True)
    c_s, s_s = _coeffs(dv_s[:, 0:32], dv_s[:, 32:64], ov_s[:, 0:32])
    dv_l = jnp.sum(Aeye, axis=1, keepdims=True)
    ov_l = jnp.sum(Aoff, axis=1, keepdims=True)
    c_l, s_l = _coeffs(dv_l[:, :, 0:32], dv_l[:, :, 32:64],
                       ov_l[:, :, 32:64])
    Top, Bot = A[:, 0:32, :], A[:, 32:64, :]
    A = jnp.concatenate([c_s * Top - s_s * Bot, s_s * Top + c_s * Bot],
                        axis=1)
    L, R = A[:, :, 0:32], A[:, :, 32:64]
    A = jnp.concatenate([c_l * L - s_l * R, s_l * L + c_l * R], axis=2)
    LV, RV = V[:, :, 0:32], V[:, :, 32:64]
    V = jnp.concatenate([c_l * LV - s_l * RV, s_l * LV + c_l * RV], axis=2)
    A = _perm_cols(_perm_rows(A))
    V = _perm_cols(V)
    return A, V


def _dense_body(S_ref, G_ref, rho_ref, qn_ref, phiS_ref, occ_ref, H_ref,
                out_ref):
    S = S_ref[...]
    G = G_ref[...]
    rho_in = rho_ref[...]
    qn = qn_ref[...]
    phiS = phiS_ref[...]
    occ = occ_ref[...]
    H = H_ref[...]

    qbasis = rho_in * S
    GOP = jnp.sum(qbasis, axis=2, keepdims=True)
    dQ = qn - GOP

    ep_list, epl_list, fockp_list = [], [], []
    for b in range(BGEOM):
        ep_b = jnp.dot(G[b], dQ[b], preferred_element_type=jnp.float32)
        epl_b = lax.dot_general(dQ[b], G[b], (((0,), (1,)), ((), ())),
                                preferred_element_type=jnp.float32)
        ep_list.append(ep_b[None])
        epl_list.append(epl_b[None])
    ep = jnp.concatenate(ep_list, axis=0)
    ep_l = jnp.concatenate(epl_list, axis=0)

    couMat = -0.5 * S * (ep + ep_l)
    F = H + couMat
    for b in range(BGEOM):
        M1 = jnp.dot(F[b], phiS[b], preferred_element_type=jnp.float32)
        f1 = _dgT00(phiS[b], M1)
        f2 = _dgT00(M1, phiS[b])
        fockp_list.append((0.5 * (f1 + f2))[None])
    A = jnp.concatenate(fockp_list, axis=0)

    eye, _ = _masks()
    V = jnp.where(eye, jnp.float32(1.0), jnp.float32(0.0))
    V = jnp.broadcast_to(V, A.shape)
    A, V = lax.fori_loop(0, 63 * NSWEEP, _jacobi_step, (A, V))

    zf = jnp.zeros_like(A)
    Aeye = jnp.where(eye, A, zf)
    d_l = jnp.sum(Aeye, axis=1, keepdims=True)
    d_s = jnp.sum(Aeye, axis=2, keepdims=True)
    il = lax.broadcasted_iota(jnp.int32, A.shape, 2)
    isub = lax.broadcasted_iota(jnp.int32, A.shape, 1)
    less = jnp.where(d_l < d_s, jnp.float32(1.0), jnp.float32(0.0))
    tie = jnp.where((d_l == d_s) & (il < isub), jnp.float32(1.0),
                    jnp.float32(0.0))
    rank = jnp.sum(less + tie, axis=2, keepdims=True)
    P = jnp.where(rank == il.astype(jnp.float32), jnp.float32(1.0),
                  jnp.float32(0.0))

    orbf_list = []
    for b in range(BGEOM):
        W_b = jnp.dot(V[b], P[b], preferred_element_type=jnp.float32)
        orb_b = jnp.dot(phiS[b], W_b, preferred_element_type=jnp.float32)
        orbf_list.append(orb_b[None])
    orb = jnp.concatenate(orbf_list, axis=0)
    orb_f = occ * orb
    rho_list = []
    for b in range(BGEOM):
        rho_list.append((2.0 * _dg11(orb_f[b], orb_f[b]))[None])
    rho = jnp.concatenate(rho_list, axis=0)

    e1 = jnp.sum(rho * H, axis=2, keepdims=True)
    e1 = jnp.sum(e1, axis=1, keepdims=True)
    e2 = 0.5 * jnp.sum(dQ * ep, axis=1, keepdims=True)
    out_ref[...] = jnp.broadcast_to(e1 + e2, (BGEOM, 1, 128))


def _mat_spec():
    return pl.BlockSpec((BGEOM, BSIZE, BSIZE), lambda g: (g, 0, 0))


def _dense_block(S, G, rho_in, qneutral, phiS, occ_mask, H):
    return pl.pallas_call(
        _dense_body,
        grid=(NGEOM // BGEOM,),
        in_specs=[_mat_spec(), _mat_spec(), _mat_spec(),
                  pl.BlockSpec((BGEOM, BSIZE, 1), lambda g: (g, 0, 0)),
                  _mat_spec(), _mat_spec(), _mat_spec()],
        out_specs=pl.BlockSpec((BGEOM, 1, 128), lambda g: (g, 0, 0)),
        out_shape=jax.ShapeDtypeStruct((NGEOM, 1, 128), jnp.float32),
    )(S, G, rho_in, qneutral, phiS, occ_mask, H)


def kernel(x, net_base, rot_tensor, S, G, rho_in, qneutral, phiS, occ_mask,
           idx_x, gather_rot, gather_oper, gather_rep, seg_rep):
    net_vals = net_base.at[idx_x].set(x)
    vals = net_vals[gather_rot].reshape((-1, 3))[:, :, None]
    rot_out_temp = jnp.matmul(rot_tensor, vals)[:, :, 0]
    rot_out = jnp.concatenate(
        [jnp.array([0.0, 1.0], dtype=jnp.float32), rot_out_temp.ravel()])
    H = rot_out[gather_oper]

    e12 = _dense_block(S, G, rho_in, qneutral, phiS, occ_mask, H)

    Erep = jax.ops.segment_sum(net_vals[gather_rep], seg_rep,
                               num_segments=NGEOM)
    return e12[:, 0, 0] + Erep


# speedup vs baseline: 2.9691x; 1.2608x over previous
import functools

import jax
import jax.numpy as jnp
from jax import lax
from jax.experimental import pallas as pl
from jax.experimental.pallas import tpu as pltpu

NGEOM = 200
BSIZE = 64
NVALS = 2000000
BGEOM = 8
NSWEEP = 4


def _dgT00(a, b):
    return lax.dot_general(a, b, (((0,), (0,)), ((), ())),
                           preferred_element_type=jnp.float32)


def _dg11(a, b):
    return lax.dot_general(a, b, (((1,), (1,)), ((), ())),
                           preferred_element_type=jnp.float32)


def _coeffs(app, aqq, apq):
    zero = apq == 0.0
    apq_s = jnp.where(zero, 1.0, apq)
    tau = (aqq - app) / (2.0 * apq_s)
    t = jnp.sign(tau) / (jnp.abs(tau) + jnp.sqrt(tau * tau + 1.0))
    t = jnp.where(tau == 0.0, 1.0, t)
    t = jnp.where(zero, 0.0, t)
    c = lax.rsqrt(t * t + 1.0)
    s = t * c
    c = jnp.where(zero, 1.0, c)
    s = jnp.where(zero, 0.0, s)
    return c, s, t


def _perm_rows(A):
    return jnp.concatenate([A[:, 0:1], A[:, 32:33], A[:, 1:31],
                            A[:, 33:64], A[:, 31:32]], axis=1)


def _perm_cols(A):
    return jnp.concatenate([A[:, :, 0:1], A[:, :, 32:33], A[:, :, 1:31],
                            A[:, :, 33:64], A[:, :, 31:32]], axis=2)


def _masks():
    ks = lax.broadcasted_iota(jnp.int32, (1, BSIZE, BSIZE), 1)
    js = lax.broadcasted_iota(jnp.int32, (1, BSIZE, BSIZE), 2)
    eye = ks == js
    off = js == ks + 32
    return eye, off


def _rot_perm_rows(A, c_s, s_s):
    T, Bt = A[:, 0:32, :], A[:, 32:64, :]
    up = c_s * T - s_s * Bt
    dn = s_s * T + c_s * Bt
    return jnp.concatenate([up[:, 0:1], dn[:, 0:1], up[:, 1:31],
                            dn[:, 1:32], up[:, 31:32]], axis=1)


def _rot_perm_cols(A, c_l, s_l):
    L, R = A[:, :, 0:32], A[:, :, 32:64]
    lf = c_l * L - s_l * R
    rt = s_l * L + c_l * R
    return jnp.concatenate([lf[:, :, 0:1], rt[:, :, 0:1], lf[:, :, 1:31],
                            rt[:, :, 1:32], lf[:, :, 31:32]], axis=2)


def _perm_lanes(d):
    return jnp.concatenate([d[:, :, 0:1], d[:, :, 32:33], d[:, :, 1:31],
                            d[:, :, 33:64], d[:, :, 31:32]], axis=2)


def _jacobi_step(_, carry):
    A, V, d = carry
    ks = lax.broadcasted_iota(jnp.int32, A.shape, 1)
    js = lax.broadcasted_iota(jnp.int32, A.shape, 2)
    off = js == ks + 32
    ov_l = jnp.sum(jnp.where(off, A, jnp.zeros_like(A)), axis=1,
                   keepdims=True)
    apq = ov_l[:, :, 32:64]
    app, aqq = d[:, :, 0:32], d[:, :, 32:64]
    c_l, s_l, t_l = _coeffs(app, aqq, apq)
    d_new = jnp.concatenate([app - t_l * apq, aqq + t_l * apq], axis=2)
    d = _perm_lanes(d_new)
    eye32 = (lax.broadcasted_iota(jnp.int32, (32, 32), 0) ==
             lax.broadcasted_iota(jnp.int32, (32, 32), 1)
             ).astype(jnp.float32)
    cs_list = []
    for b in range(BGEOM):
        CS = jnp.concatenate([c_l[b], s_l[b]], axis=0)
        CST = lax.dot_general(eye32, CS, (((1,), (1,)), ((), ())),
                              preferred_element_type=jnp.float32)
        cs_list.append(CST[None])
    CSs = jnp.concatenate(cs_list, axis=0)
    c_s, s_s = CSs[:, :, 0:1], CSs[:, :, 1:2]
    A = _rot_perm_rows(A, c_s, s_s)
    A = _rot_perm_cols(A, c_l, s_l)
    V = _rot_perm_cols(V, c_l, s_l)
    return A, V, d


def _dense_body(S_ref, G_ref, rho_ref, qn_ref, phiS_ref, occ_ref, H_ref,
                out_ref):
    S = S_ref[...]
    G = G_ref[...]
    rho_in = rho_ref[...]
    qn = qn_ref[...]
    phiS = phiS_ref[...]
    occ = occ_ref[...]
    H = H_ref[...]

    qbasis = rho_in * S
    GOP = jnp.sum(qbasis, axis=2, keepdims=True)
    dQ = qn - GOP

    ep_list, epl_list, fockp_list = [], [], []
    for b in range(BGEOM):
        ep_b = jnp.dot(G[b], dQ[b], preferred_element_type=jnp.float32)
        epl_b = lax.dot_general(dQ[b], G[b], (((0,), (1,)), ((), ())),
                                preferred_element_type=jnp.float32)
        ep_list.append(ep_b[None])
        epl_list.append(epl_b[None])
    ep = jnp.concatenate(ep_list, axis=0)
    ep_l = jnp.concatenate(epl_list, axis=0)

    couMat = -0.5 * S * (ep + ep_l)
    F = H + couMat
    for b in range(BGEOM):
        M1 = jnp.dot(F[b], phiS[b], preferred_element_type=jnp.float32)
        f1 = _dgT00(phiS[b], M1)
        f2 = _dgT00(M1, phiS[b])
        fockp_list.append((0.5 * (f1 + f2))[None])
    A = jnp.concatenate(fockp_list, axis=0)

    eye, _ = _masks()
    V = jnp.where(eye, jnp.float32(1.0), jnp.float32(0.0))
    V = jnp.broadcast_to(V, A.shape)
    zf0 = jnp.zeros_like(A)
    d0 = jnp.sum(jnp.where(eye, A, zf0), axis=1, keepdims=True)
    A, V, _ = lax.fori_loop(0, 63 * NSWEEP, _jacobi_step, (A, V, d0))

    zf = jnp.zeros_like(A)
    Aeye = jnp.where(eye, A, zf)
    d_l = jnp.sum(Aeye, axis=1, keepdims=True)
    d_s = jnp.sum(Aeye, axis=2, keepdims=True)
    il = lax.broadcasted_iota(jnp.int32, A.shape, 2)
    isub = lax.broadcasted_iota(jnp.int32, A.shape, 1)
    less = jnp.where(d_l < d_s, jnp.float32(1.0), jnp.float32(0.0))
    tie = jnp.where((d_l == d_s) & (il < isub), jnp.float32(1.0),
                    jnp.float32(0.0))
    rank = jnp.sum(less + tie, axis=2, keepdims=True)
    P = jnp.where(rank == il.astype(jnp.float32), jnp.float32(1.0),
                  jnp.float32(0.0))

    orbf_list = []
    for b in range(BGEOM):
        W_b = jnp.dot(V[b], P[b], preferred_element_type=jnp.float32)
        orb_b = jnp.dot(phiS[b], W_b, preferred_element_type=jnp.float32)
        orbf_list.append(orb_b[None])
    orb = jnp.concatenate(orbf_list, axis=0)
    orb_f = occ * orb
    rho_list = []
    for b in range(BGEOM):
        rho_list.append((2.0 * _dg11(orb_f[b], orb_f[b]))[None])
    rho = jnp.concatenate(rho_list, axis=0)

    e1 = jnp.sum(rho * H, axis=2, keepdims=True)
    e1 = jnp.sum(e1, axis=1, keepdims=True)
    e2 = 0.5 * jnp.sum(dQ * ep, axis=1, keepdims=True)
    out_ref[...] = jnp.broadcast_to(e1 + e2, (BGEOM, 1, 128))


def _mat_spec():
    return pl.BlockSpec((BGEOM, BSIZE, BSIZE), lambda g: (g, 0, 0))


def _dense_block(S, G, rho_in, qneutral, phiS, occ_mask, H):
    return pl.pallas_call(
        _dense_body,
        grid=(NGEOM // BGEOM,),
        in_specs=[_mat_spec(), _mat_spec(), _mat_spec(),
                  pl.BlockSpec((BGEOM, BSIZE, 1), lambda g: (g, 0, 0)),
                  _mat_spec(), _mat_spec(), _mat_spec()],
        out_specs=pl.BlockSpec((BGEOM, 1, 128), lambda g: (g, 0, 0)),
        out_shape=jax.ShapeDtypeStruct((NGEOM, 1, 128), jnp.float32),
    )(S, G, rho_in, qneutral, phiS, occ_mask, H)


def kernel(x, net_base, rot_tensor, S, G, rho_in, qneutral, phiS, occ_mask,
           idx_x, gather_rot, gather_oper, gather_rep, seg_rep):
    net_vals = net_base.at[idx_x].set(x)
    vals = net_vals[gather_rot].reshape((-1, 3))[:, :, None]
    rot_out_temp = jnp.matmul(rot_tensor, vals)[:, :, 0]
    rot_out = jnp.concatenate(
        [jnp.array([0.0, 1.0], dtype=jnp.float32), rot_out_temp.ravel()])
    H = rot_out[gather_oper]

    e12 = _dense_block(S, G, rho_in, qneutral, phiS, occ_mask, H)

    Erep = jax.ops.segment_sum(net_vals[gather_rep], seg_rep,
                               num_segments=NGEOM)
    return e12[:, 0, 0] + Erep


# speedup vs baseline: 3.3965x; 1.1440x over previous
import functools

import jax
import jax.numpy as jnp
from jax import lax
from jax.experimental import pallas as pl
from jax.experimental.pallas import tpu as pltpu

NGEOM = 200
BSIZE = 64
NVALS = 2000000
BGEOM = 16
NSWEEP = 4


def _dgT00(a, b):
    return lax.dot_general(a, b, (((0,), (0,)), ((), ())),
                           preferred_element_type=jnp.float32)


def _dg11(a, b):
    return lax.dot_general(a, b, (((1,), (1,)), ((), ())),
                           preferred_element_type=jnp.float32)


def _coeffs(app, aqq, apq):
    zero = apq == 0.0
    apq_s = jnp.where(zero, 1.0, apq)
    tau = (aqq - app) / (2.0 * apq_s)
    t = jnp.sign(tau) / (jnp.abs(tau) + jnp.sqrt(tau * tau + 1.0))
    t = jnp.where(tau == 0.0, 1.0, t)
    t = jnp.where(zero, 0.0, t)
    c = lax.rsqrt(t * t + 1.0)
    s = t * c
    c = jnp.where(zero, 1.0, c)
    s = jnp.where(zero, 0.0, s)
    return c, s, t


def _perm_rows(A):
    return jnp.concatenate([A[:, 0:1], A[:, 32:33], A[:, 1:31],
                            A[:, 33:64], A[:, 31:32]], axis=1)


def _perm_cols(A):
    return jnp.concatenate([A[:, :, 0:1], A[:, :, 32:33], A[:, :, 1:31],
                            A[:, :, 33:64], A[:, :, 31:32]], axis=2)


def _masks():
    ks = lax.broadcasted_iota(jnp.int32, (1, BSIZE, BSIZE), 1)
    js = lax.broadcasted_iota(jnp.int32, (1, BSIZE, BSIZE), 2)
    eye = ks == js
    off = js == ks + 32
    return eye, off


def _rot_perm_rows(A, c_s, s_s):
    T, Bt = A[:, 0:32, :], A[:, 32:64, :]
    up = c_s * T - s_s * Bt
    dn = s_s * T + c_s * Bt
    return jnp.concatenate([up[:, 0:1], dn[:, 0:1], up[:, 1:31],
                            dn[:, 1:32], up[:, 31:32]], axis=1)


def _rot_perm_cols(A, c_l, s_l):
    L, R = A[:, :, 0:32], A[:, :, 32:64]
    lf = c_l * L - s_l * R
    rt = s_l * L + c_l * R
    return jnp.concatenate([lf[:, :, 0:1], rt[:, :, 0:1], lf[:, :, 1:31],
                            rt[:, :, 1:32], lf[:, :, 31:32]], axis=2)


def _perm_lanes(d):
    return jnp.concatenate([d[:, :, 0:1], d[:, :, 32:33], d[:, :, 1:31],
                            d[:, :, 33:64], d[:, :, 31:32]], axis=2)


def _jacobi_step(_, carry):
    A, V, d = carry
    ks = lax.broadcasted_iota(jnp.int32, A.shape, 1)
    js = lax.broadcasted_iota(jnp.int32, A.shape, 2)
    off = js == ks + 32
    ov_l = jnp.sum(jnp.where(off, A, jnp.zeros_like(A)), axis=1,
                   keepdims=True)
    apq = ov_l[:, :, 32:64]
    app, aqq = d[:, :, 0:32], d[:, :, 32:64]
    c_l, s_l, t_l = _coeffs(app, aqq, apq)
    d_new = jnp.concatenate([app - t_l * apq, aqq + t_l * apq], axis=2)
    d = _perm_lanes(d_new)
    eye32 = (lax.broadcasted_iota(jnp.int32, (32, 32), 0) ==
             lax.broadcasted_iota(jnp.int32, (32, 32), 1)
             ).astype(jnp.float32)
    cs_list = []
    for b in range(BGEOM):
        CS = jnp.concatenate([c_l[b], s_l[b]], axis=0)
        CST = lax.dot_general(eye32, CS, (((1,), (1,)), ((), ())),
                              preferred_element_type=jnp.float32)
        cs_list.append(CST[None])
    CSs = jnp.concatenate(cs_list, axis=0)
    c_s, s_s = CSs[:, :, 0:1], CSs[:, :, 1:2]
    A = _rot_perm_rows(A, c_s, s_s)
    A = _rot_perm_cols(A, c_l, s_l)
    V = _rot_perm_cols(V, c_l, s_l)
    return A, V, d


def _dense_body(S_ref, G_ref, rho_ref, qn_ref, phiS_ref, occ_ref, H_ref,
                out_ref):
    S = S_ref[...]
    G = G_ref[...]
    rho_in = rho_ref[...]
    qn = qn_ref[...]
    phiS = phiS_ref[...]
    occ = occ_ref[...]
    H = H_ref[...]

    qbasis = rho_in * S
    GOP = jnp.sum(qbasis, axis=2, keepdims=True)
    dQ = qn - GOP

    ep_list, epl_list, fockp_list = [], [], []
    for b in range(BGEOM):
        ep_b = jnp.dot(G[b], dQ[b], preferred_element_type=jnp.float32)
        epl_b = lax.dot_general(dQ[b], G[b], (((0,), (1,)), ((), ())),
                                preferred_element_type=jnp.float32)
        ep_list.append(ep_b[None])
        epl_list.append(epl_b[None])
    ep = jnp.concatenate(ep_list, axis=0)
    ep_l = jnp.concatenate(epl_list, axis=0)

    couMat = -0.5 * S * (ep + ep_l)
    F = H + couMat
    for b in range(BGEOM):
        M1 = jnp.dot(F[b], phiS[b], preferred_element_type=jnp.float32)
        f1 = _dgT00(phiS[b], M1)
        f2 = _dgT00(M1, phiS[b])
        fockp_list.append((0.5 * (f1 + f2))[None])
    A = jnp.concatenate(fockp_list, axis=0)

    eye, _ = _masks()
    V = jnp.where(eye, jnp.float32(1.0), jnp.float32(0.0))
    V = jnp.broadcast_to(V, A.shape)
    zf0 = jnp.zeros_like(A)
    d0 = jnp.sum(jnp.where(eye, A, zf0), axis=1, keepdims=True)
    A, V, _ = lax.fori_loop(0, 63 * NSWEEP, _jacobi_step, (A, V, d0))

    zf = jnp.zeros_like(A)
    Aeye = jnp.where(eye, A, zf)
    d_l = jnp.sum(Aeye, axis=1, keepdims=True)
    d_s = jnp.sum(Aeye, axis=2, keepdims=True)
    il = lax.broadcasted_iota(jnp.int32, A.shape, 2)
    isub = lax.broadcasted_iota(jnp.int32, A.shape, 1)
    less = jnp.where(d_l < d_s, jnp.float32(1.0), jnp.float32(0.0))
    tie = jnp.where((d_l == d_s) & (il < isub), jnp.float32(1.0),
                    jnp.float32(0.0))
    rank = jnp.sum(less + tie, axis=2, keepdims=True)
    P = jnp.where(rank == il.astype(jnp.float32), jnp.float32(1.0),
                  jnp.float32(0.0))

    orbf_list = []
    for b in range(BGEOM):
        W_b = jnp.dot(V[b], P[b], preferred_element_type=jnp.float32)
        orb_b = jnp.dot(phiS[b], W_b, preferred_element_type=jnp.float32)
        orbf_list.append(orb_b[None])
    orb = jnp.concatenate(orbf_list, axis=0)
    orb_f = occ * orb
    rho_list = []
    for b in range(BGEOM):
        rho_list.append((2.0 * _dg11(orb_f[b], orb_f[b]))[None])
    rho = jnp.concatenate(rho_list, axis=0)

    e1 = jnp.sum(rho * H, axis=2, keepdims=True)
    e1 = jnp.sum(e1, axis=1, keepdims=True)
    e2 = 0.5 * jnp.sum(dQ * ep, axis=1, keepdims=True)
    out_ref[...] = jnp.broadcast_to(e1 + e2, (BGEOM, 1, 128))


def _mat_spec():
    return pl.BlockSpec((BGEOM, BSIZE, BSIZE), lambda g: (g, 0, 0))


def _dense_block(S, G, rho_in, qneutral, phiS, occ_mask, H):
    return pl.pallas_call(
        _dense_body,
        grid=(NGEOM // BGEOM,),
        in_specs=[_mat_spec(), _mat_spec(), _mat_spec(),
                  pl.BlockSpec((BGEOM, BSIZE, 1), lambda g: (g, 0, 0)),
                  _mat_spec(), _mat_spec(), _mat_spec()],
        out_specs=pl.BlockSpec((BGEOM, 1, 128), lambda g: (g, 0, 0)),
        out_shape=jax.ShapeDtypeStruct((NGEOM, 1, 128), jnp.float32),
    )(S, G, rho_in, qneutral, phiS, occ_mask, H)


def kernel(x, net_base, rot_tensor, S, G, rho_in, qneutral, phiS, occ_mask,
           idx_x, gather_rot, gather_oper, gather_rep, seg_rep):
    net_vals = net_base.at[idx_x].set(x)
    vals = net_vals[gather_rot].reshape((-1, 3))[:, :, None]
    rot_out_temp = jnp.matmul(rot_tensor, vals)[:, :, 0]
    rot_out = jnp.concatenate(
        [jnp.array([0.0, 1.0], dtype=jnp.float32), rot_out_temp.ravel()])
    H = rot_out[gather_oper]

    e12 = _dense_block(S, G, rho_in, qneutral, phiS, occ_mask, H)

    Erep = jax.ops.segment_sum(net_vals[gather_rep], seg_rep,
                               num_segments=NGEOM)
    return e12[:, 0, 0] + Erep
